# baseline (speedup 1.0000x reference)
import jax
import jax.numpy as jnp
from jax.experimental import pallas as pl
from jax.experimental.pallas import tpu as pltpu

B = 16
T = 512
IN_DIM = 512
HID = 256
G7 = 7 * HID
TB = 64
K = T // TB


def _ghnn_block(x_ref, dt_ref, sl_ref, wx_ref, wh_ref, b_ref, out_ref,
                h_ref, c_ref, ct_ref, gx_ref, oacc_ref, xt_ref):
    k = pl.program_id(0)
    brow = b_ref[...]
    wx = wx_ref[...]

    xt_ref[...] = jnp.swapaxes(x_ref[...], 0, 1)

    @pl.when(k == 0)
    def _prologue():
        h_ref[...] = jnp.zeros_like(h_ref)
        c_ref[...] = jnp.zeros_like(c_ref)
        ct_ref[...] = jnp.zeros_like(ct_ref)
        xb = xt_ref[...].reshape(TB * B, IN_DIM)
        gx_ref[0] = jnp.dot(
            xb.astype(jnp.bfloat16), wx,
            preferred_element_type=jnp.float32) + brow

    @pl.when(k > 0)
    def _scan():
        wh = wh_ref[...]
        sl = sl_ref[...]
        t0 = (k - 1) * TB
        cbuf = jax.lax.rem(k - 1, 2)
        nbuf = jax.lax.rem(k, 2)

        def step(t, carry):
            h, c, ct = carry
            g = gx_ref[cbuf, pl.ds(t * B, B), :]
            g = g + jnp.dot(h.astype(jnp.bfloat16), wh,
                            preferred_element_type=jnp.float32)
            sig = jax.nn.sigmoid(g[:, :5 * HID])
            ig = sig[:, 0:HID]
            fg = sig[:, HID:2 * HID]
            og = sig[:, 2 * HID:3 * HID]
            itg = sig[:, 3 * HID:4 * HID]
            ftg = sig[:, 4 * HID:5 * HID]
            z = jnp.tanh(g[:, 5 * HID:6 * HID])
            decay = jax.nn.softplus(g[:, 6 * HID:7 * HID])
            dtt = dt_ref[t]
            c_i = fg * c + ig * z
            ct_new = ftg * ct + itg * z
            c_new = ct_new + (c_i - ct_new) * jnp.exp(-decay * dtt)
            h_new = og * jnp.tanh(c_new)
            mask = (t0 + t) < sl
            h2 = jnp.where(mask, h_new, h)
            oacc_ref[t] = h2

            @pl.when(k < K)
            def _fused_gx():
                xrow = xt_ref[t].astype(jnp.bfloat16)
                gx_ref[nbuf, pl.ds(t * B, B), :] = jnp.dot(
                    xrow, wx, preferred_element_type=jnp.float32) + brow

            return (h2, c_new, ct_new)

        h, c, ct = jax.lax.fori_loop(
            0, TB, step, (h_ref[...], c_ref[...], ct_ref[...]), unroll=4)
        h_ref[...] = h
        c_ref[...] = c
        ct_ref[...] = ct
        out_ref[...] = jnp.swapaxes(oacc_ref[...], 0, 1)


def _run(x, dt3, sl, Wx, Wh, brow, interpret=False):
    return pl.pallas_call(
        _ghnn_block,
        grid=(K + 1,),
        in_specs=[
            pl.BlockSpec((B, TB, IN_DIM),
                         lambda k: (0, jnp.minimum(k, K - 1), 0)),
            pl.BlockSpec((TB, B, 1),
                         lambda k: (jnp.maximum(k - 1, 0), 0, 0)),
            pl.BlockSpec((B, 1), lambda k: (0, 0)),
            pl.BlockSpec((IN_DIM, G7), lambda k: (0, 0)),
            pl.BlockSpec((HID, G7), lambda k: (0, 0)),
            pl.BlockSpec((1, G7), lambda k: (0, 0)),
        ],
        out_specs=pl.BlockSpec((B, TB, HID),
                               lambda k: (0, jnp.maximum(k - 1, 0), 0)),
        out_shape=jax.ShapeDtypeStruct((B, T, HID), jnp.float32),
        scratch_shapes=[
            pltpu.VMEM((B, HID), jnp.float32),
            pltpu.VMEM((B, HID), jnp.float32),
            pltpu.VMEM((B, HID), jnp.float32),
            pltpu.VMEM((2, TB * B, G7), jnp.float32),
            pltpu.VMEM((TB, B, HID), jnp.float32),
            pltpu.VMEM((TB, B, IN_DIM), jnp.float32),
        ],
        compiler_params=pltpu.CompilerParams(
            dimension_semantics=("arbitrary",)),
        interpret=interpret,
    )(x, dt3, sl, Wx, Wh, brow)


def kernel(x, dt, seq_lens, W_ig, b_ig, W_fg, b_fg, W_og, b_og,
           W_it, b_it, W_ft, b_ft, W_z, b_z, W_d, b_d):
    dt3 = jnp.swapaxes(dt, 0, 1)[:, :, None]
    sl = seq_lens.astype(jnp.int32).reshape(B, 1)
    Ws = (W_ig, W_fg, W_og, W_it, W_ft, W_z, W_d)
    bs = (b_ig, b_fg, b_og, b_it, b_ft, b_z, b_d)
    Wx = jnp.concatenate(
        [W[:IN_DIM] for W in Ws], axis=1).astype(jnp.bfloat16)
    Wh = jnp.concatenate(
        [W[IN_DIM:] for W in Ws], axis=1).astype(jnp.bfloat16)
    brow = jnp.concatenate(bs).reshape(1, G7)
    return _run(x, dt3, sl, Wx, Wh, brow)


# speedup vs baseline: 1.8680x; 1.8680x over previous
import jax
import jax.numpy as jnp
from jax.experimental import pallas as pl
from jax.experimental.pallas import tpu as pltpu

B = 16
T = 512
IN_DIM = 512
HID = 256
G7 = 7 * HID
TB = 128


def _ghnn_block(x_ref, dt_ref, sl_ref, wx_ref, wh_ref, b_ref, out_ref,
                h_ref, c_ref, ct_ref, gx_ref, oacc_ref):
    k = pl.program_id(0)

    @pl.when(k == 0)
    def _init():
        h_ref[...] = jnp.zeros_like(h_ref)
        c_ref[...] = jnp.zeros_like(c_ref)
        ct_ref[...] = jnp.zeros_like(ct_ref)

    xb = jnp.swapaxes(x_ref[...], 0, 1).reshape(TB * B, IN_DIM)
    gx_ref[...] = jnp.dot(
        xb.astype(jnp.bfloat16), wx_ref[...],
        preferred_element_type=jnp.float32) + b_ref[...]
    wh = wh_ref[...]

    sl = sl_ref[...]
    t0 = k * TB

    def step(t, carry):
        h, c, ct = carry
        hb = h.astype(jnp.bfloat16)
        g0 = jnp.dot(hb[:, :HID // 2], wh[:HID // 2],
                     preferred_element_type=jnp.float32)
        g1 = jnp.dot(hb[:, HID // 2:], wh[HID // 2:],
                     preferred_element_type=jnp.float32)
        g = gx_ref[pl.ds(t * B, B), :] + g0 + g1
        sig = jax.nn.sigmoid(g[:, :5 * HID])
        ig = sig[:, 0:HID]
        fg = sig[:, HID:2 * HID]
        og = sig[:, 2 * HID:3 * HID]
        itg = sig[:, 3 * HID:4 * HID]
        ftg = sig[:, 4 * HID:5 * HID]
        z = jnp.tanh(g[:, 5 * HID:6 * HID])
        decay = jax.nn.softplus(g[:, 6 * HID:7 * HID])
        dtt = dt_ref[t]
        c_i = fg * c + ig * z
        ct_new = ftg * ct + itg * z
        c_new = ct_new + (c_i - ct_new) * jnp.exp(-decay * dtt)
        h_new = og * jnp.tanh(c_new)
        mask = (t0 + t) < sl
        h2 = jnp.where(mask, h_new, h)
        oacc_ref[t] = h2
        return (h2, c_new, ct_new)

    h, c, ct = jax.lax.fori_loop(
        0, TB, step, (h_ref[...], c_ref[...], ct_ref[...]), unroll=8)
    h_ref[...] = h
    c_ref[...] = c
    ct_ref[...] = ct
    out_ref[...] = jnp.swapaxes(oacc_ref[...], 0, 1)


def _run(x, dt3, sl, Wx, Wh, brow, interpret=False):
    return pl.pallas_call(
        _ghnn_block,
        grid=(T // TB,),
        in_specs=[
            pl.BlockSpec((B, TB, IN_DIM), lambda k: (0, k, 0)),
            pl.BlockSpec((TB, B, 1), lambda k: (k, 0, 0)),
            pl.BlockSpec((B, 1), lambda k: (0, 0)),
            pl.BlockSpec((IN_DIM, G7), lambda k: (0, 0)),
            pl.BlockSpec((HID, G7), lambda k: (0, 0)),
            pl.BlockSpec((1, G7), lambda k: (0, 0)),
        ],
        out_specs=pl.BlockSpec((B, TB, HID), lambda k: (0, k, 0)),
        out_shape=jax.ShapeDtypeStruct((B, T, HID), jnp.float32),
        scratch_shapes=[
            pltpu.VMEM((B, HID), jnp.float32),
            pltpu.VMEM((B, HID), jnp.float32),
            pltpu.VMEM((B, HID), jnp.float32),
            pltpu.VMEM((TB * B, G7), jnp.float32),
            pltpu.VMEM((TB, B, HID), jnp.float32),
        ],
        compiler_params=pltpu.CompilerParams(
            dimension_semantics=("arbitrary",)),
        interpret=interpret,
    )(x, dt3, sl, Wx, Wh, brow)


def kernel(x, dt, seq_lens, W_ig, b_ig, W_fg, b_fg, W_og, b_og,
           W_it, b_it, W_ft, b_ft, W_z, b_z, W_d, b_d):
    dt3 = jnp.swapaxes(dt, 0, 1)[:, :, None]
    sl = seq_lens.astype(jnp.int32).reshape(B, 1)
    Ws = (W_ig, W_fg, W_og, W_it, W_ft, W_z, W_d)
    bs = (b_ig, b_fg, b_og, b_it, b_ft, b_z, b_d)
    Wx = jnp.concatenate(
        [W[:IN_DIM] for W in Ws], axis=1).astype(jnp.bfloat16)
    Wh = jnp.concatenate(
        [W[IN_DIM:] for W in Ws], axis=1).astype(jnp.bfloat16)
    brow = jnp.concatenate(bs).reshape(1, G7)
    return _run(x, dt3, sl, Wx, Wh, brow)


# speedup vs baseline: 1.9360x; 1.0364x over previous
import jax
import jax.numpy as jnp
from jax.experimental import pallas as pl
from jax.experimental.pallas import tpu as pltpu

B = 16
T = 512
IN_DIM = 512
HID = 256
G7 = 7 * HID
TB = 128


def _ghnn_block(x_ref, dt_ref, sl_ref, wx_ref, wh_ref, b_ref, out_ref,
                h_ref, c_ref, ct_ref, gx_ref, oacc_ref):
    k = pl.program_id(0)

    @pl.when(k == 0)
    def _init():
        h_ref[...] = jnp.zeros_like(h_ref)
        c_ref[...] = jnp.zeros_like(c_ref)
        ct_ref[...] = jnp.zeros_like(ct_ref)

    xb = jnp.swapaxes(x_ref[...], 0, 1).reshape(TB * B, IN_DIM)
    gx_ref[...] = jnp.dot(
        xb.astype(jnp.bfloat16), wx_ref[...],
        preferred_element_type=jnp.float32) + b_ref[...]
    wh = wh_ref[...]

    sl = sl_ref[...]
    t0 = k * TB

    def step(t, carry):
        h, c, ct = carry
        g = gx_ref[pl.ds(t * B, B), :]
        g = g + jnp.dot(h.astype(jnp.bfloat16), wh,
                        preferred_element_type=jnp.float32)
        sig = jax.nn.sigmoid(g[:, :5 * HID])
        ig = sig[:, 0:HID]
        fg = sig[:, HID:2 * HID]
        og = sig[:, 2 * HID:3 * HID]
        itg = sig[:, 3 * HID:4 * HID]
        ftg = sig[:, 4 * HID:5 * HID]
        z = jnp.tanh(g[:, 5 * HID:6 * HID])
        decay = jax.nn.softplus(g[:, 6 * HID:7 * HID])
        dtt = dt_ref[t]
        c_i = fg * c + ig * z
        ct_new = ftg * ct + itg * z
        c_new = ct_new + (c_i - ct_new) * jnp.exp(-decay * dtt)
        h_new = og * jnp.tanh(c_new)
        mask = (t0 + t) < sl
        h2 = jnp.where(mask, h_new, h)
        oacc_ref[t] = h2
        return (h2, c_new, ct_new)

    h, c, ct = jax.lax.fori_loop(
        0, TB, step, (h_ref[...], c_ref[...], ct_ref[...]), unroll=16)
    h_ref[...] = h
    c_ref[...] = c
    ct_ref[...] = ct
    out_ref[...] = jnp.swapaxes(oacc_ref[...], 0, 1)


def _run(x, dt3, sl, Wx, Wh, brow, interpret=False):
    return pl.pallas_call(
        _ghnn_block,
        grid=(T // TB,),
        in_specs=[
            pl.BlockSpec((B, TB, IN_DIM), lambda k: (0, k, 0)),
            pl.BlockSpec((TB, B, 1), lambda k: (k, 0, 0)),
            pl.BlockSpec((B, 1), lambda k: (0, 0)),
            pl.BlockSpec((IN_DIM, G7), lambda k: (0, 0)),
            pl.BlockSpec((HID, G7), lambda k: (0, 0)),
            pl.BlockSpec((1, G7), lambda k: (0, 0)),
        ],
        out_specs=pl.BlockSpec((B, TB, HID), lambda k: (0, k, 0)),
        out_shape=jax.ShapeDtypeStruct((B, T, HID), jnp.float32),
        scratch_shapes=[
            pltpu.VMEM((B, HID), jnp.float32),
            pltpu.VMEM((B, HID), jnp.float32),
            pltpu.VMEM((B, HID), jnp.float32),
            pltpu.VMEM((TB * B, G7), jnp.float32),
            pltpu.VMEM((TB, B, HID), jnp.float32),
        ],
        compiler_params=pltpu.CompilerParams(
            dimension_semantics=("arbitrary",)),
        interpret=interpret,
    )(x, dt3, sl, Wx, Wh, brow)


def kernel(x, dt, seq_lens, W_ig, b_ig, W_fg, b_fg, W_og, b_og,
           W_it, b_it, W_ft, b_ft, W_z, b_z, W_d, b_d):
    dt3 = jnp.swapaxes(dt, 0, 1)[:, :, None]
    sl = seq_lens.astype(jnp.int32).reshape(B, 1)
    Ws = (W_ig, W_fg, W_og, W_it, W_ft, W_z, W_d)
    bs = (b_ig, b_fg, b_og, b_it, b_ft, b_z, b_d)
    Wx = jnp.concatenate(
        [W[:IN_DIM] for W in Ws], axis=1).astype(jnp.bfloat16)
    Wh = jnp.concatenate(
        [W[IN_DIM:] for W in Ws], axis=1).astype(jnp.bfloat16)
    brow = jnp.concatenate(bs).reshape(1, G7)
    return _run(x, dt3, sl, Wx, Wh, brow)
